# baseline (speedup 1.0000x reference)
import functools

import jax
import jax.numpy as jnp
from jax import lax
from jax.experimental import pallas as pl
from jax.experimental.pallas import tpu as pltpu
from jax.experimental.pallas import tpu_sc as plsc

NC, NS = 2, 16
NW = NC * NS
CHUNK = 128
NB = 4
L = 16


@functools.partial(jax.jit, static_argnums=(2, 3, 4))
def _sc_gather_t(tok2d, table, s0, s1, D):
    n_chunks = tok2d.shape[0]
    cpw = n_chunks // NW
    bpj = s0 // CHUNK
    assert cpw % NB == 0 and cpw // NB >= 3
    n_grps = cpw // NB
    mesh = plsc.VectorSubcoreMesh(core_axis_name="c", subcore_axis_name="s")

    @functools.partial(
        pl.kernel,
        out_type=jax.ShapeDtypeStruct((s1, D, s0), jnp.float32),
        mesh=mesh,
        scratch_types=[
            pltpu.VMEM((cpw, CHUNK), jnp.int32),
            pltpu.VMEM((NB, CHUNK, D), jnp.float32),
            pltpu.VMEM((NB, D, CHUNK), jnp.float32),
            pltpu.SemaphoreType.DMA,
            pltpu.SemaphoreType.DMA,
        ],
        compiler_params=pltpu.CompilerParams(
            use_tc_tiling_on_sc=False, needs_layout_passes=False),
    )
    def k(tok_hbm, table_hbm, out_hbm, idx_v, bufs, bufTs, sem_g, sem_w):
        wid = lax.axis_index("s") * NC + lax.axis_index("c")
        c0 = wid * cpw
        pltpu.sync_copy(tok_hbm.at[pl.ds(c0, cpw)], idx_v)

        i16 = lax.iota(jnp.int32, 16)
        rows = [i16 + L * kk for kk in range(CHUNK // L)]

        def start_gather(t, b):
            pltpu.async_copy(table_hbm.at[idx_v.at[t]], bufs.at[b], sem_g)

        def wait_gather():
            pltpu.make_async_copy(
                table_hbm.at[idx_v.at[0]], bufs.at[0], sem_g).wait()

        def start_write(t, b):
            cid = c0 + t
            j = cid // bpj
            b0 = (cid % bpj) * CHUNK
            pltpu.async_copy(
                bufTs.at[b], out_hbm.at[j, :, pl.ds(b0, CHUNK)], sem_w)

        def wait_write():
            pltpu.make_async_copy(
                bufTs.at[0], out_hbm.at[0, :, pl.ds(0, CHUNK)], sem_w).wait()

        def transpose(b):
            src = bufs.at[b]
            for d in range(D):
                col = jnp.full((16,), d, jnp.int32)
                for kk in range(CHUNK // L):
                    v = plsc.load_gather(src, [rows[kk], col])
                    bufTs[b, d, pl.ds(kk * L, L)] = v

        for b in range(NB):
            start_gather(b, b)

        def chunk(t, carry):
            b = t & (NB - 1)
            wait_gather()

            @pl.when(t >= NB)
            def _():
                wait_write()

            transpose(b)
            start_write(t, b)

            @pl.when(t + NB < cpw)
            def _():
                start_gather(t + NB, b)

            return carry

        lax.fori_loop(0, cpw, chunk, 0, unroll=False)

        for _ in range(NB):
            wait_write()

    return k(tok2d, table)


def kernel(token_ids, embedding_lookup):
    s0, s1 = token_ids.shape
    D = embedding_lookup.shape[1]
    tok2d = token_ids.T.reshape((s0 * s1) // CHUNK, CHUNK)
    outP = _sc_gather_t(tok2d, embedding_lookup, s0, s1, D)
    return outP.transpose(2, 0, 1)


# speedup vs baseline: 1.1565x; 1.1565x over previous
import functools

import jax
import jax.numpy as jnp
from jax import lax
from jax.experimental import pallas as pl
from jax.experimental.pallas import tpu as pltpu
from jax.experimental.pallas import tpu_sc as plsc

NC, NS = 2, 16
NW = NC * NS
CHUNK = 128
QW = 2
WCHUNK = CHUNK * QW
L = 16


@functools.partial(jax.jit, static_argnums=(2, 3, 4))
def _sc_gather_t(tok2d, table, s0, s1, D):
    n_chunks = tok2d.shape[0]
    cpw = n_chunks // NW
    wpw = cpw // QW
    wpj = s0 // WCHUNK
    assert cpw % QW == 0 and wpw >= 3 and s0 % WCHUNK == 0
    mesh = plsc.VectorSubcoreMesh(core_axis_name="c", subcore_axis_name="s")

    @functools.partial(
        pl.kernel,
        out_type=jax.ShapeDtypeStruct((s1, D, s0), jnp.float32),
        mesh=mesh,
        scratch_types=[
            pltpu.VMEM((cpw, CHUNK), jnp.int32),
            pltpu.VMEM((QW, CHUNK, D), jnp.float32),
            pltpu.VMEM((2, D, WCHUNK), jnp.float32),
            pltpu.SemaphoreType.DMA,
            pltpu.SemaphoreType.DMA,
        ],
        compiler_params=pltpu.CompilerParams(
            use_tc_tiling_on_sc=False, needs_layout_passes=False),
    )
    def k(tok_hbm, table_hbm, out_hbm, idx_v, bufs, bufTs, sem_g, sem_w):
        wid = lax.axis_index("s") * NC + lax.axis_index("c")
        c0 = wid * cpw
        w0 = wid * wpw
        pltpu.sync_copy(tok_hbm.at[pl.ds(c0, cpw)], idx_v)

        i16 = lax.iota(jnp.int32, 16)
        rows = [i16 + L * kk for kk in range(CHUNK // L)]

        def start_gather(t, q):
            pltpu.async_copy(table_hbm.at[idx_v.at[t]], bufs.at[q], sem_g)

        def wait_gather():
            pltpu.make_async_copy(
                table_hbm.at[idx_v.at[0]], bufs.at[0], sem_g).wait()

        def start_write(w, wb):
            gw = w0 + w
            j = gw // wpj
            b0 = (gw % wpj) * WCHUNK
            pltpu.async_copy(
                bufTs.at[wb], out_hbm.at[j, :, pl.ds(b0, WCHUNK)], sem_w)

        def wait_write():
            pltpu.make_async_copy(
                bufTs.at[0], out_hbm.at[0, :, pl.ds(0, WCHUNK)], sem_w).wait()

        def transpose(q, wb):
            src = bufs.at[q]
            for d0 in range(0, D, 2):
                vals = []
                for d in (d0, d0 + 1):
                    col = jnp.full((16,), d, jnp.int32)
                    for kk in range(CHUNK // L):
                        vals.append(plsc.load_gather(src, [rows[kk], col]))
                for i, v in enumerate(vals):
                    d = d0 + i // (CHUNK // L)
                    kk = i % (CHUNK // L)
                    bufTs[wb, d, pl.ds(q * CHUNK + kk * L, L)] = v

        for q in range(QW):
            start_gather(q, q)

        def slab(w, carry):
            wb = w & 1

            @pl.when(w >= 2)
            def _():
                wait_write()

            for q in range(QW):
                wait_gather()
                transpose(q, wb)

                @pl.when(w + 1 < wpw)
                def _():
                    start_gather((w + 1) * QW + q, q)

            start_write(w, wb)
            return carry

        lax.fori_loop(0, wpw, slab, 0, unroll=False)

        for _ in range(2):
            wait_write()

    return k(tok2d, table)


def kernel(token_ids, embedding_lookup):
    s0, s1 = token_ids.shape
    D = embedding_lookup.shape[1]
    tok2d = token_ids.T.reshape((s0 * s1) // CHUNK, CHUNK)
    outP = _sc_gather_t(tok2d, embedding_lookup, s0, s1, D)
    return outP.transpose(2, 0, 1)


# speedup vs baseline: 1.7579x; 1.5200x over previous
import functools

import jax
import jax.numpy as jnp
from jax import lax
from jax.experimental import pallas as pl
from jax.experimental.pallas import tpu as pltpu
from jax.experimental.pallas import tpu_sc as plsc

NC, NS = 2, 16
NW = NC * NS
CHUNK = 128
QW = 2
WCHUNK = CHUNK * QW
L = 16


@functools.partial(jax.jit, static_argnums=(2, 3, 4))
def _sc_gather_t(tok2d, table, s0, s1, D):
    n_chunks = tok2d.shape[0]
    cpw = n_chunks // NW
    wpw = cpw // QW
    wpj = s0 // WCHUNK
    assert cpw % QW == 0 and wpw >= 3 and s0 % WCHUNK == 0
    mesh = plsc.VectorSubcoreMesh(core_axis_name="c", subcore_axis_name="s")

    @functools.partial(
        pl.kernel,
        out_type=jax.ShapeDtypeStruct((s1, D, s0), jnp.float32),
        mesh=mesh,
        scratch_types=[
            pltpu.VMEM((cpw, CHUNK), jnp.int32),
            pltpu.VMEM((QW, CHUNK, D), jnp.float32),
            pltpu.VMEM((2, D, WCHUNK), jnp.float32),
            pltpu.SemaphoreType.DMA,
            pltpu.SemaphoreType.DMA,
        ],
        compiler_params=pltpu.CompilerParams(
            use_tc_tiling_on_sc=False, needs_layout_passes=False),
    )
    def k(tok_hbm, table_hbm, out_hbm, idx_v, bufs, bufTs, sem_g, sem_w):
        wid = lax.axis_index("s") * NC + lax.axis_index("c")
        c0 = wid * cpw
        w0 = wid * wpw
        pltpu.sync_copy(tok_hbm.at[pl.ds(c0, cpw)], idx_v)

        i16 = lax.iota(jnp.int32, 16)
        rows = [i16 + L * kk for kk in range(CHUNK // L)]

        def start_gather(t, q):
            pltpu.async_copy(table_hbm.at[idx_v.at[t]], bufs.at[q], sem_g)

        def wait_gather():
            pltpu.make_async_copy(
                table_hbm.at[idx_v.at[0]], bufs.at[0], sem_g).wait()

        def start_write(w, wb):
            gw = w0 + w
            j = gw // wpj
            b0 = (gw % wpj) * WCHUNK
            pltpu.async_copy(
                bufTs.at[wb], out_hbm.at[j, :, pl.ds(b0, WCHUNK)], sem_w)

        def wait_write():
            pltpu.make_async_copy(
                bufTs.at[0], out_hbm.at[0, :, pl.ds(0, WCHUNK)], sem_w).wait()

        def transpose(q, wb):
            src = bufs.at[q]
            dst = bufTs.at[wb]

            def kbody(kk, carry):
                row_v = i16 + kk * L
                col_v = i16 + (q * CHUNK + kk * L)
                for d0 in range(0, D, 8):
                    work = []
                    for d in range(d0, d0 + 8):
                        diag = (i16 + d) & (D - 1)
                        work.append(
                            (diag, plsc.load_gather(src, [row_v, diag])))
                    for diag, v in work:
                        plsc.store_scatter(dst, [diag, col_v], v)
                return carry

            lax.fori_loop(0, CHUNK // L, kbody, 0, unroll=False)

        for q in range(QW):
            start_gather(q, q)

        def slab(w, carry):
            wb = w & 1

            @pl.when(w >= 2)
            def _():
                wait_write()

            for q in range(QW):
                wait_gather()
                transpose(q, wb)

                @pl.when(w + 1 < wpw)
                def _():
                    start_gather((w + 1) * QW + q, q)

            start_write(w, wb)
            return carry

        lax.fori_loop(0, wpw, slab, 0, unroll=False)

        for _ in range(2):
            wait_write()

    return k(tok2d, table)


def kernel(token_ids, embedding_lookup):
    s0, s1 = token_ids.shape
    D = embedding_lookup.shape[1]
    tok2d = token_ids.T.reshape((s0 * s1) // CHUNK, CHUNK)
    outP = _sc_gather_t(tok2d, embedding_lookup, s0, s1, D)
    return outP.transpose(2, 0, 1)


# speedup vs baseline: 2.2169x; 1.2611x over previous
import functools

import jax
import jax.numpy as jnp
from jax import lax
from jax.experimental import pallas as pl
from jax.experimental.pallas import tpu as pltpu
from jax.experimental.pallas import tpu_sc as plsc

NC, NS = 2, 16
NW = NC * NS
CHUNK = 128
NB = 2
L = 16


@functools.partial(jax.jit, static_argnums=(3, 4, 5))
def _sc_gather_t(tok2d, pair2d, table2, s0, s1, D):
    n_chunks = tok2d.shape[0]
    cpw = n_chunks // NW
    bpj = s0 // CHUNK
    assert cpw % NB == 0 and cpw // NB >= 3
    mesh = plsc.VectorSubcoreMesh(core_axis_name="c", subcore_axis_name="s")

    @functools.partial(
        pl.kernel,
        out_type=jax.ShapeDtypeStruct((s1, D, s0), jnp.float32),
        mesh=mesh,
        scratch_types=[
            pltpu.VMEM((cpw, CHUNK), jnp.int32),
            pltpu.VMEM((cpw, CHUNK), jnp.int32),
            pltpu.VMEM((NB, CHUNK, 2 * D), jnp.float32),
            pltpu.VMEM((2, D, CHUNK), jnp.float32),
            pltpu.SemaphoreType.DMA,
            pltpu.SemaphoreType.DMA,
        ],
        compiler_params=pltpu.CompilerParams(needs_layout_passes=False),
    )
    def k(tok_hbm, pair_hbm, table_hbm, out_hbm,
          tok_v, pair_v, bufs, bufTs, sem_g, sem_w):
        wid = lax.axis_index("s") * NC + lax.axis_index("c")
        c0 = wid * cpw
        pltpu.sync_copy(tok_hbm.at[pl.ds(c0, cpw)], tok_v)
        pltpu.sync_copy(pair_hbm.at[pl.ds(c0, cpw)], pair_v)

        i16 = lax.iota(jnp.int32, 16)

        def start_gather(t, b):
            pltpu.async_copy(table_hbm.at[pair_v.at[t]], bufs.at[b], sem_g)

        def wait_gather():
            pltpu.make_async_copy(
                table_hbm.at[pair_v.at[0]], bufs.at[0], sem_g).wait()

        def start_write(t, wb):
            cid = c0 + t
            j = cid // bpj
            b0 = (cid % bpj) * CHUNK
            pltpu.async_copy(
                bufTs.at[wb], out_hbm.at[j, :, pl.ds(b0, CHUNK)], sem_w)

        def wait_write():
            pltpu.make_async_copy(
                bufTs.at[0], out_hbm.at[0, :, pl.ds(0, CHUNK)], sem_w).wait()

        def transpose(t, b, wb):
            src = bufs.at[b]
            dst = bufTs.at[wb]

            def kbody(kk, carry):
                row_v = i16 + kk * L
                col_v = row_v
                par = (tok_v[t, pl.ds(kk * L, L)] & 1) * D
                for d0 in range(0, D, 8):
                    work = []
                    for d in range(d0, d0 + 8):
                        diag = (i16 + d) & (D - 1)
                        work.append(
                            (diag,
                             plsc.load_gather(src, [row_v, par + diag])))
                    for diag, v in work:
                        plsc.store_scatter(dst, [diag, col_v], v)
                return carry

            lax.fori_loop(0, CHUNK // L, kbody, 0, unroll=False)

        for b in range(NB):
            start_gather(b, b)

        def chunk(t, carry):
            b = t & (NB - 1)
            wb = t & 1
            wait_gather()

            @pl.when(t >= 2)
            def _():
                wait_write()

            transpose(t, b, wb)
            start_write(t, wb)

            @pl.when(t + NB < cpw)
            def _():
                start_gather(t + NB, b)

            return carry

        lax.fori_loop(0, cpw, chunk, 0, unroll=False)

        for _ in range(2):
            wait_write()

    return k(tok2d, pair2d, table2)


def kernel(token_ids, embedding_lookup):
    s0, s1 = token_ids.shape
    V, D = embedding_lookup.shape
    tokT = token_ids.T
    tok2d = tokT.reshape((s0 * s1) // CHUNK, CHUNK)
    pair2d = (tokT >> 1).reshape((s0 * s1) // CHUNK, CHUNK)
    table2 = embedding_lookup.reshape(V // 2, 2 * D)
    outP = _sc_gather_t(tok2d, pair2d, table2, s0, s1, D)
    return outP.transpose(2, 0, 1)


# speedup vs baseline: 2.4902x; 1.1233x over previous
import functools

import jax
import jax.numpy as jnp
from jax import lax
from jax.experimental import pallas as pl
from jax.experimental.pallas import tpu as pltpu
from jax.experimental.pallas import tpu_sc as plsc

NC, NS = 2, 16
NW = NC * NS
CHUNK = 128
NB = 4
L = 16


@functools.partial(jax.jit, static_argnums=(2, 3, 4))
def _sc_gather_t(tok2d, tablep, s0, s1, D):
    n_chunks = tok2d.shape[0]
    cpw = n_chunks // NW
    bpj = s0 // CHUNK
    assert cpw % NB == 0 and cpw // NB >= 3
    mesh = plsc.VectorSubcoreMesh(core_axis_name="c", subcore_axis_name="s")

    @functools.partial(
        pl.kernel,
        out_type=jax.ShapeDtypeStruct((s1, D, s0), jnp.float32),
        mesh=mesh,
        scratch_types=[
            pltpu.VMEM((cpw, CHUNK), jnp.int32),
            pltpu.VMEM((NB, CHUNK, 2 * D), jnp.float32),
            pltpu.VMEM((2, D, CHUNK), jnp.float32),
            pltpu.SemaphoreType.DMA,
            pltpu.SemaphoreType.DMA,
        ],
        compiler_params=pltpu.CompilerParams(needs_layout_passes=False),
    )
    def k(tok_hbm, table_hbm, out_hbm,
          tok_v, bufs, bufTs, sem_g, sem_w):
        wid = lax.axis_index("s") * NC + lax.axis_index("c")
        c0 = wid * cpw
        pltpu.sync_copy(tok_hbm.at[pl.ds(c0, cpw)], tok_v)

        i16 = lax.iota(jnp.int32, 16)

        def start_gather(t, b):
            pltpu.async_copy(table_hbm.at[tok_v.at[t]], bufs.at[b], sem_g)

        def wait_gather():
            pltpu.make_async_copy(
                table_hbm.at[tok_v.at[0]], bufs.at[0], sem_g).wait()

        def start_write(t, wb):
            cid = c0 + t
            j = cid // bpj
            b0 = (cid % bpj) * CHUNK
            pltpu.async_copy(
                bufTs.at[wb], out_hbm.at[j, :, pl.ds(b0, CHUNK)], sem_w)

        def wait_write():
            pltpu.make_async_copy(
                bufTs.at[0], out_hbm.at[0, :, pl.ds(0, CHUNK)], sem_w).wait()

        def transpose(t, b, wb):
            src = bufs.at[b]
            dst = bufTs.at[wb]

            def kbody(kk, carry):
                row_v = i16 + kk * L
                col_v = row_v
                for d0 in range(0, D, 8):
                    work = []
                    for d in range(d0, d0 + 8):
                        diag = (i16 + d) & (D - 1)
                        work.append(
                            (diag,
                             plsc.load_gather(src, [row_v, diag])))
                    for diag, v in work:
                        plsc.store_scatter(dst, [diag, col_v], v)
                return carry

            lax.fori_loop(0, CHUNK // L, kbody, 0, unroll=False)

        for b in range(NB):
            start_gather(b, b)

        def chunk(t, carry):
            b = t & (NB - 1)
            wb = t & 1
            wait_gather()

            @pl.when(t >= 2)
            def _():
                wait_write()

            transpose(t, b, wb)
            start_write(t, wb)

            @pl.when(t + NB < cpw)
            def _():
                start_gather(t + NB, b)

            return carry

        lax.fori_loop(0, cpw, chunk, 0, unroll=False)

        for _ in range(2):
            wait_write()

    return k(tok2d, tablep)


def kernel(token_ids, embedding_lookup):
    s0, s1 = token_ids.shape
    V, D = embedding_lookup.shape
    tokT = token_ids.T
    tok2d = tokT.reshape((s0 * s1) // CHUNK, CHUNK)
    tablep = jnp.pad(embedding_lookup, ((0, 0), (0, D)))
    outP = _sc_gather_t(tok2d, tablep, s0, s1, D)
    return outP.transpose(2, 0, 1)
